# baseline (speedup 1.0000x reference)
import functools

import jax
import jax.numpy as jnp
from jax import lax
from jax.experimental import pallas as pl
from jax.experimental.pallas import tpu as pltpu
from jax.experimental.pallas import tpu_sc as plsc

N_ROWS = 16384
N_COLS = 1000
LANES = 16
NUM_CORES = 2
NUM_SUBCORES = 16
NUM_WORKERS = NUM_CORES * NUM_SUBCORES
ROWS_PER_W = N_ROWS // NUM_WORKERS
BLK_ROWS = 32
BLK_ELEMS = BLK_ROWS * N_COLS
N_BLKS = ROWS_PER_W // BLK_ROWS
GRPS_PER_BLK = BLK_ROWS // LANES
LAB_PAD = 1024


def _wtac_body(dist_hbm, labels_hbm, theta_hbm, out_hbm,
               labels_v, theta_v, buf0, buf1, outb, sem0, sem1):
    wid = lax.axis_index("s") * NUM_CORES + lax.axis_index("c")
    base = wid * ROWS_PER_W * N_COLS

    pltpu.sync_copy(labels_hbm, labels_v)
    pltpu.sync_copy(theta_hbm, theta_v)
    theta_vec = theta_v[...]

    lmax = labels_v[pl.ds(0, LANES)]
    for i in range(1, LAB_PAD // LANES):
        lmax = jnp.maximum(lmax, labels_v[pl.ds(i * LANES, LANES)])
    row_iota = lax.iota(jnp.int32, LANES)
    dnums = lax.GatherDimensionNumbers(
        offset_dims=(), collapsed_slice_dims=(0,), start_index_map=(0,))
    for sh in (8, 4, 2, 1):
        perm = jnp.bitwise_and(row_iota + sh, LANES - 1)
        shuffled = lax.gather(
            lmax, perm[:, None], dnums, slice_sizes=(1,),
            mode=lax.GatherScatterMode.PROMISE_IN_BOUNDS)
        lmax = jnp.maximum(lmax, shuffled)
    outlier_vec = lmax + 1

    row_off = row_iota * N_COLS
    bufs = (buf0, buf1)
    sems = (sem0, sem1)

    def start(blk):
        return pltpu.async_copy(
            dist_hbm.at[pl.ds(base + blk * BLK_ELEMS, BLK_ELEMS)],
            bufs[blk & 1], sems[blk & 1])

    pending = start(0)
    for blk in range(N_BLKS):
        nxt = start(blk + 1) if blk + 1 < N_BLKS else None
        pending.wait()
        buf = bufs[blk & 1]
        for grp in range(GRPS_PER_BLK):
            idx0 = row_off + (grp * LANES * N_COLS)

            def body(_, carry, _buf=buf):
                mv, mi, idx = carry
                v = plsc.load_gather(_buf, [idx])
                pred = v < mv
                return (jnp.where(pred, v, mv),
                        jnp.where(pred, idx, mi),
                        idx + 1)

            mv, mi, _ = lax.fori_loop(
                0, N_COLS, body,
                (jnp.full((LANES,), jnp.inf, jnp.float32),
                 idx0, idx0),
                unroll=8)
            col = mi - idx0
            lab = plsc.load_gather(labels_v, [col])
            res = jnp.where(mv < theta_vec, lab, outlier_vec)
            outb[pl.ds(blk * BLK_ROWS + grp * LANES, LANES)] = res
        pending = nxt

    pltpu.sync_copy(outb, out_hbm.at[pl.ds(wid * ROWS_PER_W, ROWS_PER_W)])


_wtac = functools.partial(
    pl.kernel,
    mesh=plsc.VectorSubcoreMesh(core_axis_name="c", subcore_axis_name="s"),
    out_type=jax.ShapeDtypeStruct((N_ROWS,), jnp.int32),
    compiler_params=pltpu.CompilerParams(needs_layout_passes=False),
    scratch_types=[
        pltpu.VMEM((LAB_PAD,), jnp.int32),
        pltpu.VMEM((LANES,), jnp.float32),
        pltpu.VMEM((BLK_ELEMS,), jnp.float32),
        pltpu.VMEM((BLK_ELEMS,), jnp.float32),
        pltpu.VMEM((ROWS_PER_W,), jnp.int32),
        pltpu.SemaphoreType.DMA,
        pltpu.SemaphoreType.DMA,
    ],
)(_wtac_body)


def kernel(distances, labels, theta_boundary):
    labels32 = labels.astype(jnp.int32)
    labels_p = jnp.concatenate(
        [labels32, jnp.broadcast_to(labels32[:1], (LAB_PAD - N_COLS,))])
    theta_vec = jnp.broadcast_to(
        jnp.asarray(theta_boundary, jnp.float32), (LANES,))
    return _wtac(distances.reshape(-1), labels_p, theta_vec)


# speedup vs baseline: 1.0039x; 1.0039x over previous
import functools

import jax
import jax.numpy as jnp
from jax import lax
from jax.experimental import pallas as pl
from jax.experimental.pallas import tpu as pltpu
from jax.experimental.pallas import tpu_sc as plsc

N_ROWS = 16384
N_COLS = 1000
LANES = 16
NUM_CORES = 2
NUM_SUBCORES = 16
NUM_WORKERS = NUM_CORES * NUM_SUBCORES
ROWS_PER_W = N_ROWS // NUM_WORKERS
BLK_ROWS = 32
N_BLKS = ROWS_PER_W // BLK_ROWS
GRPS_PER_BLK = BLK_ROWS // LANES
VROW = 2 * N_COLS
VROWS_PER_BLK = BLK_ROWS // 2
LAB_PAD = 1024


def _wtac_body(dist_hbm, labels_hbm, theta_hbm, out_hbm,
               labels_v, theta_v, buf0, buf1, outb, sem0, sem1):
    wid = lax.axis_index("s") * NUM_CORES + lax.axis_index("c")

    pltpu.sync_copy(labels_hbm, labels_v)
    pltpu.sync_copy(theta_hbm, theta_v)
    theta_vec = theta_v[...]

    lmax = labels_v[pl.ds(0, LANES)]
    for i in range(1, LAB_PAD // LANES):
        lmax = jnp.maximum(lmax, labels_v[pl.ds(i * LANES, LANES)])
    row_iota = lax.iota(jnp.int32, LANES)
    dnums = lax.GatherDimensionNumbers(
        offset_dims=(), collapsed_slice_dims=(0,), start_index_map=(0,))
    for sh in (8, 4, 2, 1):
        perm = jnp.bitwise_and(row_iota + sh, LANES - 1)
        shuffled = lax.gather(
            lmax, perm[:, None], dnums, slice_sizes=(1,),
            mode=lax.GatherScatterMode.PROMISE_IN_BOUNDS)
        lmax = jnp.maximum(lmax, shuffled)
    outlier_vec = lmax + 1

    bufs = (buf0, buf1)
    sems = (sem0, sem1)
    vrow_base = wid * N_BLKS * VROWS_PER_BLK

    def start(blk):
        vrows = vrow_base + (blk * VROWS_PER_BLK) + row_iota
        return pltpu.async_copy(dist_hbm.at[vrows], bufs[blk & 1],
                                sems[blk & 1])

    scol0 = jnp.bitwise_and(row_iota, 1) * N_COLS

    pending = start(0)
    for blk in range(N_BLKS):
        nxt = start(blk + 1) if blk + 1 < N_BLKS else None
        pending.wait()
        buf = bufs[blk & 1]
        for grp in range(GRPS_PER_BLK):
            srow_g = jnp.full((LANES,), grp * (LANES // 2), jnp.int32) + \
                jnp.right_shift(row_iota, 1)

            def body(_, carry, _buf=buf, _srow=srow_g):
                mv, mi, idxc = carry
                v = plsc.load_gather(_buf, [_srow, idxc])
                pred = v < mv
                return (jnp.where(pred, v, mv),
                        jnp.where(pred, idxc, mi),
                        idxc + 1)

            mv, mi, _ = lax.fori_loop(
                0, N_COLS, body,
                (jnp.full((LANES,), jnp.inf, jnp.float32),
                 scol0, scol0),
                unroll=8)
            col = mi - scol0
            lab = plsc.load_gather(labels_v, [col])
            res = jnp.where(mv < theta_vec, lab, outlier_vec)
            outb[pl.ds(blk * BLK_ROWS + grp * LANES, LANES)] = res
        pending = nxt

    pltpu.sync_copy(outb, out_hbm.at[pl.ds(wid * ROWS_PER_W, ROWS_PER_W)])


_wtac = functools.partial(
    pl.kernel,
    mesh=plsc.VectorSubcoreMesh(core_axis_name="c", subcore_axis_name="s"),
    out_type=jax.ShapeDtypeStruct((N_ROWS,), jnp.int32),
    compiler_params=pltpu.CompilerParams(
        needs_layout_passes=False, use_tc_tiling_on_sc=False),
    scratch_types=[
        pltpu.VMEM((LAB_PAD,), jnp.int32),
        pltpu.VMEM((LANES,), jnp.float32),
        pltpu.VMEM((VROWS_PER_BLK, VROW), jnp.float32),
        pltpu.VMEM((VROWS_PER_BLK, VROW), jnp.float32),
        pltpu.VMEM((ROWS_PER_W,), jnp.int32),
        pltpu.SemaphoreType.DMA,
        pltpu.SemaphoreType.DMA,
    ],
)(_wtac_body)


def kernel(distances, labels, theta_boundary):
    labels32 = labels.astype(jnp.int32)
    labels_p = jnp.concatenate(
        [labels32, jnp.broadcast_to(labels32[:1], (LAB_PAD - N_COLS,))])
    theta_vec = jnp.broadcast_to(
        jnp.asarray(theta_boundary, jnp.float32), (LANES,))
    return _wtac(distances.reshape(N_ROWS // 2, VROW), labels_p, theta_vec)


# speedup vs baseline: 1.1239x; 1.1196x over previous
import functools

import jax
import jax.numpy as jnp
from jax import lax
from jax.experimental import pallas as pl
from jax.experimental.pallas import tpu as pltpu
from jax.experimental.pallas import tpu_sc as plsc

N_ROWS = 16384
N_COLS = 1000
LANES = 16
NUM_CORES = 2
NUM_SUBCORES = 16
NUM_WORKERS = NUM_CORES * NUM_SUBCORES
ROWS_PER_W = N_ROWS // NUM_WORKERS
BLK_ROWS = 32
N_BLKS = ROWS_PER_W // BLK_ROWS
VROW = 2 * N_COLS
VROWS_PER_BLK = BLK_ROWS // 2
LAB_PAD = 1024
N_CHUNKS = N_COLS // LANES
TAIL0 = N_COLS - LANES
SUB = 4
PITCH = LANES + 1


def _wtac_body(dist_hbm, labels_hbm, theta_hbm, out_hbm,
               labels_v, theta_v, buf0, buf1, outb, mvs_s, mis_s,
               sem0, sem1):
    wid = lax.axis_index("s") * NUM_CORES + lax.axis_index("c")

    pltpu.sync_copy(labels_hbm, labels_v)
    pltpu.sync_copy(theta_hbm, theta_v)
    theta_vec = theta_v[...]

    lmax = labels_v[pl.ds(0, LANES)]
    for i in range(1, LAB_PAD // LANES):
        lmax = jnp.maximum(lmax, labels_v[pl.ds(i * LANES, LANES)])
    lane = lax.iota(jnp.int32, LANES)
    dnums = lax.GatherDimensionNumbers(
        offset_dims=(), collapsed_slice_dims=(0,), start_index_map=(0,))
    for sh in (8, 4, 2, 1):
        perm = jnp.bitwise_and(lane + sh, LANES - 1)
        shuffled = lax.gather(
            lmax, perm[:, None], dnums, slice_sizes=(1,),
            mode=lax.GatherScatterMode.PROMISE_IN_BOUNDS)
        lmax = jnp.maximum(lmax, shuffled)
    outlier_vec = lmax + 1

    lane17 = lane * PITCH
    inf16 = jnp.full((LANES,), jnp.inf, jnp.float32)
    cid_tail = lane + TAIL0
    vrow_base = wid * N_BLKS * VROWS_PER_BLK

    def start(blk, par):
        vrows = vrow_base + (blk * VROWS_PER_BLK) + lane
        return pltpu.async_copy(dist_hbm.at[vrows],
                                (buf0, buf1)[par], (sem0, sem1)[par])

    def wait(par):
        buf = (buf0, buf1)[par]
        sem = (sem0, sem1)[par]
        pltpu.make_async_copy(dist_hbm.at[vrow_base + lane], buf, sem).wait()

    def compute_block(buf, blk):
        for grp in range(2):
            for sb in range(LANES // SUB):
                rows = [grp * LANES + sb * SUB + r for r in range(SUB)]
                offs = [(row >> 1, (row & 1) * N_COLS) for row in rows]

                def body(k, carry, _offs=tuple(offs)):
                    mvs = list(carry[0])
                    mis = list(carry[1])
                    cid = carry[2]
                    koff = k * LANES
                    for r in range(SUB):
                        vr, co = _offs[r]
                        v = buf[vr, pl.ds(koff + co, LANES)]
                        pred = v < mvs[r]
                        mvs[r] = jnp.where(pred, v, mvs[r])
                        mis[r] = jnp.where(pred, cid, mis[r])
                    return (tuple(mvs), tuple(mis), cid + LANES)

                mvs, mis, _ = lax.fori_loop(
                    0, N_CHUNKS, body,
                    ((inf16,) * SUB, (lane,) * SUB, lane), unroll=2)
                mvs, mis = list(mvs), list(mis)
                for r in range(SUB):
                    vr, co = offs[r]
                    v = buf[vr, pl.ds(TAIL0 + co, LANES)]
                    pred = v < mvs[r]
                    mvs[r] = jnp.where(pred, v, mvs[r])
                    mis[r] = jnp.where(pred, cid_tail, mis[r])
                    sl = sb * SUB + r
                    plsc.store_scatter(mvs_s, [lane + sl * PITCH], mvs[r])
                    plsc.store_scatter(mis_s, [lane + sl * PITCH], mis[r])

            bm = plsc.load_gather(mvs_s, [lane17])
            bmi = plsc.load_gather(mis_s, [lane17])
            for k in range(1, LANES):
                v = plsc.load_gather(mvs_s, [lane17 + k])
                vi = plsc.load_gather(mis_s, [lane17 + k])
                better = (v < bm) | ((v == bm) & (vi < bmi))
                bm = jnp.where(better, v, bm)
                bmi = jnp.where(better, vi, bmi)
            lab = plsc.load_gather(labels_v, [bmi])
            res = jnp.where(bm < theta_vec, lab, outlier_vec)
            outb[pl.ds(blk * BLK_ROWS + grp * LANES, LANES)] = res

    start(0, 0)
    start(1, 1)

    def pair(p, carry):
        blk = p * 2
        wait(0)
        compute_block(buf0, blk)

        @pl.when(p < (N_BLKS // 2) - 1)
        def _():
            start(blk + 2, 0)

        wait(1)
        compute_block(buf1, blk + 1)

        @pl.when(p < (N_BLKS // 2) - 1)
        def _():
            start(blk + 3, 1)

        return carry

    lax.fori_loop(0, N_BLKS // 2, pair, 0)

    pltpu.sync_copy(outb, out_hbm.at[pl.ds(wid * ROWS_PER_W, ROWS_PER_W)])


_wtac = functools.partial(
    pl.kernel,
    mesh=plsc.VectorSubcoreMesh(core_axis_name="c", subcore_axis_name="s"),
    out_type=jax.ShapeDtypeStruct((N_ROWS,), jnp.int32),
    compiler_params=pltpu.CompilerParams(
        needs_layout_passes=False, use_tc_tiling_on_sc=False),
    scratch_types=[
        pltpu.VMEM((LAB_PAD,), jnp.int32),
        pltpu.VMEM((LANES,), jnp.float32),
        pltpu.VMEM((VROWS_PER_BLK, VROW), jnp.float32),
        pltpu.VMEM((VROWS_PER_BLK, VROW), jnp.float32),
        pltpu.VMEM((ROWS_PER_W,), jnp.int32),
        pltpu.VMEM((LANES * PITCH,), jnp.float32),
        pltpu.VMEM((LANES * PITCH,), jnp.int32),
        pltpu.SemaphoreType.DMA,
        pltpu.SemaphoreType.DMA,
    ],
)(_wtac_body)


def kernel(distances, labels, theta_boundary):
    labels32 = labels.astype(jnp.int32)
    labels_p = jnp.concatenate(
        [labels32, jnp.broadcast_to(labels32[:1], (LAB_PAD - N_COLS,))])
    theta_vec = jnp.broadcast_to(
        jnp.asarray(theta_boundary, jnp.float32), (LANES,))
    return _wtac(distances.reshape(N_ROWS // 2, VROW), labels_p, theta_vec)


# speedup vs baseline: 1.7738x; 1.5782x over previous
import functools

import jax
import jax.numpy as jnp
from jax import lax
from jax.experimental import pallas as pl
from jax.experimental.pallas import tpu as pltpu
from jax.experimental.pallas import tpu_sc as plsc

N_ROWS = 16384
N_COLS = 1000
LANES = 16
NUM_CORES = 2
NUM_SUBCORES = 16
NUM_WORKERS = NUM_CORES * NUM_SUBCORES
ROWS_PER_W = N_ROWS // NUM_WORKERS
BLK_ROWS = 32
N_BLKS = ROWS_PER_W // BLK_ROWS
VROW = 2 * N_COLS
VROWS_PER_BLK = BLK_ROWS // 2
LAB_PAD = 1024
N_CHUNKS = N_COLS // LANES
TAIL0 = N_COLS - LANES
SUB = 4
PITCH = LANES + 1


def _wtac_body(dist_hbm, labels_hbm, theta_hbm, out_hbm,
               labels_v, theta_v, buf0, buf1, outb, mvs_s, mis_s,
               sem0, sem1):
    wid = lax.axis_index("s") * NUM_CORES + lax.axis_index("c")

    pltpu.sync_copy(labels_hbm, labels_v)
    pltpu.sync_copy(theta_hbm, theta_v)
    theta_vec = theta_v[...]

    lmax = labels_v[pl.ds(0, LANES)]
    for i in range(1, LAB_PAD // LANES):
        lmax = jnp.maximum(lmax, labels_v[pl.ds(i * LANES, LANES)])
    lane = lax.iota(jnp.int32, LANES)
    dnums = lax.GatherDimensionNumbers(
        offset_dims=(), collapsed_slice_dims=(0,), start_index_map=(0,))
    for sh in (8, 4, 2, 1):
        perm = jnp.bitwise_and(lane + sh, LANES - 1)
        shuffled = lax.gather(
            lmax, perm[:, None], dnums, slice_sizes=(1,),
            mode=lax.GatherScatterMode.PROMISE_IN_BOUNDS)
        lmax = jnp.maximum(lmax, shuffled)
    outlier_vec = lmax + 1

    lane17 = lane * PITCH
    inf16 = jnp.full((LANES,), jnp.inf, jnp.float32)
    cid_tail = lane + TAIL0
    row_base = wid * ROWS_PER_W

    def start(blk, par):
        return pltpu.async_copy(
            dist_hbm.at[pl.ds(row_base + blk * BLK_ROWS, BLK_ROWS)],
            (buf0, buf1)[par], (sem0, sem1)[par])

    def wait(par):
        buf = (buf0, buf1)[par]
        sem = (sem0, sem1)[par]
        pltpu.make_async_copy(dist_hbm.at[pl.ds(0, BLK_ROWS)], buf,
                              sem).wait()

    def compute_block(buf, blk):
        for grp in range(2):
            for sb in range(LANES // SUB):
                rows = [grp * LANES + sb * SUB + r for r in range(SUB)]

                def body(k, carry, _rows=tuple(rows)):
                    mvs = list(carry[0])
                    mis = list(carry[1])
                    cid = carry[2]
                    koff = k * LANES
                    for r in range(SUB):
                        v = buf[_rows[r], pl.ds(koff, LANES)]
                        pred = v < mvs[r]
                        mvs[r] = jnp.where(pred, v, mvs[r])
                        mis[r] = jnp.where(pred, cid, mis[r])
                    return (tuple(mvs), tuple(mis), cid + LANES)

                mvs, mis, _ = lax.fori_loop(
                    0, N_CHUNKS, body,
                    ((inf16,) * SUB, (lane,) * SUB, lane), unroll=2)
                mvs, mis = list(mvs), list(mis)
                for r in range(SUB):
                    v = buf[rows[r], pl.ds(TAIL0, LANES)]
                    pred = v < mvs[r]
                    mvs[r] = jnp.where(pred, v, mvs[r])
                    mis[r] = jnp.where(pred, cid_tail, mis[r])
                    sl = sb * SUB + r
                    plsc.store_scatter(mvs_s, [lane + sl * PITCH], mvs[r])
                    plsc.store_scatter(mis_s, [lane + sl * PITCH], mis[r])

            bm = plsc.load_gather(mvs_s, [lane17])
            bmi = plsc.load_gather(mis_s, [lane17])
            for k in range(1, LANES):
                v = plsc.load_gather(mvs_s, [lane17 + k])
                vi = plsc.load_gather(mis_s, [lane17 + k])
                better = (v < bm) | ((v == bm) & (vi < bmi))
                bm = jnp.where(better, v, bm)
                bmi = jnp.where(better, vi, bmi)
            lab = plsc.load_gather(labels_v, [bmi])
            res = jnp.where(bm < theta_vec, lab, outlier_vec)
            outb[pl.ds(blk * BLK_ROWS + grp * LANES, LANES)] = res

    start(0, 0)
    start(1, 1)

    def pair(p, carry):
        blk = p * 2
        wait(0)
        compute_block(buf0, blk)

        @pl.when(p < (N_BLKS // 2) - 1)
        def _():
            start(blk + 2, 0)

        wait(1)
        compute_block(buf1, blk + 1)

        @pl.when(p < (N_BLKS // 2) - 1)
        def _():
            start(blk + 3, 1)

        return carry

    lax.fori_loop(0, N_BLKS // 2, pair, 0)

    pltpu.sync_copy(outb, out_hbm.at[pl.ds(wid * ROWS_PER_W, ROWS_PER_W)])


_wtac = functools.partial(
    pl.kernel,
    mesh=plsc.VectorSubcoreMesh(core_axis_name="c", subcore_axis_name="s"),
    out_type=jax.ShapeDtypeStruct((N_ROWS,), jnp.int32),
    compiler_params=pltpu.CompilerParams(
        needs_layout_passes=False, use_tc_tiling_on_sc=True),
    scratch_types=[
        pltpu.VMEM((LAB_PAD,), jnp.int32),
        pltpu.VMEM((LANES,), jnp.float32),
        pltpu.VMEM((BLK_ROWS, N_COLS), jnp.float32),
        pltpu.VMEM((BLK_ROWS, N_COLS), jnp.float32),
        pltpu.VMEM((ROWS_PER_W,), jnp.int32),
        pltpu.VMEM((LANES * PITCH,), jnp.float32),
        pltpu.VMEM((LANES * PITCH,), jnp.int32),
        pltpu.SemaphoreType.DMA,
        pltpu.SemaphoreType.DMA,
    ],
)(_wtac_body)


def kernel(distances, labels, theta_boundary):
    labels32 = labels.astype(jnp.int32)
    labels_p = jnp.concatenate(
        [labels32, jnp.broadcast_to(labels32[:1], (LAB_PAD - N_COLS,))])
    theta_vec = jnp.broadcast_to(
        jnp.asarray(theta_boundary, jnp.float32), (LANES,))
    return _wtac(distances, labels_p, theta_vec)
